# baseline (speedup 1.0000x reference)
import functools

import jax
import jax.numpy as jnp
from jax import lax
from jax.experimental import pallas as pl
from jax.experimental.pallas import tpu as pltpu
from jax.experimental.pallas import tpu_sc as plsc

N = 10000
E = 320000
D = 128
DP = 144
NW = 32
EPW = E // NW
C = 80
NCH = EPW // C
RPT = N // 16


def _featp_and_er(f, alT, arT):
    n = f.shape[0]
    el = jnp.dot(f, alT, preferred_element_type=jnp.float32,
                 precision=lax.Precision.HIGHEST)
    er = jnp.dot(f, arT, preferred_element_type=jnp.float32,
                 precision=lax.Precision.HIGHEST)
    featp = jnp.concatenate(
        [f, jnp.ones((n, 1), jnp.float32), el,
         jnp.zeros((n, DP - D - 2), jnp.float32)], axis=1)
    return featp, er


def _tc_feat_body(x_ref, w_ref, alT_ref, arT_ref, featp_ref, er_ref):
    f = jnp.dot(x_ref[...], w_ref[...], preferred_element_type=jnp.float32,
                precision=lax.Precision.HIGHEST)
    featp_ref[...], er_ref[...] = _featp_and_er(f, alT_ref[...], arT_ref[...])


_tc_feat = pl.pallas_call(
    _tc_feat_body,
    out_shape=[
        jax.ShapeDtypeStruct((N, DP), jnp.float32),
        jax.ShapeDtypeStruct((N, 1), jnp.float32),
    ],
)


def _tc_mid_body(acc_ref, b_ref, w_ref, alT_ref, arT_ref, featp_ref, er_ref):
    a = acc_ref[0] + acc_ref[1]
    den = a[:, D:D + 1]
    den = jnp.where(den == 0.0, 1.0, den)
    h = jnp.maximum(a[:, :D] / den + b_ref[...], 0.0)
    f = jnp.dot(h, w_ref[...], preferred_element_type=jnp.float32,
                precision=lax.Precision.HIGHEST)
    featp_ref[...], er_ref[...] = _featp_and_er(f, alT_ref[...], arT_ref[...])


_tc_mid = pl.pallas_call(
    _tc_mid_body,
    out_shape=[
        jax.ShapeDtypeStruct((N, DP), jnp.float32),
        jax.ShapeDtypeStruct((N, 1), jnp.float32),
    ],
)


def _tc_out_body(acc_ref, b_ref, out_ref):
    a = acc_ref[0] + acc_ref[1]
    den = a[:, D:D + 1]
    den = jnp.where(den == 0.0, 1.0, den)
    out_ref[...] = a[:, :D] / den + b_ref[...]


_tc_out = pl.pallas_call(
    _tc_out_body,
    out_shape=jax.ShapeDtypeStruct((N, D), jnp.float32),
)


NPASS = 5
CPP = NCH // NPASS
NBUF = 3


def _sc_gat_body(featp_hbm, er_hbm, src_hbm, dst_hbm, out_hbm,
                 srcp_v, dstp_v, rows0, rows1, rows2, ers0, ers1, ers2, ex_v,
                 acc_sh, semr0, semr1, semr2, sere0, sere1, sere2,
                 sems0, sems1, sems2):
    cid = lax.axis_index("c")
    sid = lax.axis_index("s")
    wid = cid * 16 + sid

    rows = [rows0, rows1, rows2]
    ers = [ers0, ers1, ers2]
    semr = [semr0, semr1, semr2]
    sere = [sere0, sere1, sere2]
    sems = [sems0, sems1, sems2]

    def issue_gathers(slot, j):
        pltpu.async_copy(er_hbm.at[dstp_v.at[j]], ers[slot], sere[slot])
        pltpu.async_copy(featp_hbm.at[srcp_v.at[j]], rows[slot], semr[slot])

    def wait_rows(slot):
        pltpu.make_async_copy(featp_hbm.at[pl.ds(0, C)], rows[slot],
                              semr[slot]).wait()

    def wait_ers(slot):
        pltpu.make_async_copy(er_hbm.at[pl.ds(0, C)], ers[slot],
                              sere[slot]).wait()

    def wait_scatter(slot):
        pltpu.make_async_copy(featp_hbm.at[pl.ds(0, C)], rows[slot],
                              sems[slot]).wait()

    def compute_and_scatter(slot, j):
        lane = jnp.arange(16, dtype=jnp.int32)
        col = jnp.full((16,), D + 1, jnp.int32)
        wait_ers(slot)
        wait_rows(slot)
        rv = rows[slot]
        for g in range(C // 16):
            el16 = plsc.load_gather(rv, [g * 16 + lane, col])
            er16 = ers[slot][pl.ds(g * 16, 16)]
            e = el16 + er16
            e = jnp.maximum(e, 0.2 * e)
            ex_v[pl.ds(g * 16, 16)] = jnp.exp(e)

        @plsc.parallel_loop(0, C, unroll=4)
        def _(r):
            a = plsc.load_gather(ex_v, [jnp.full((16,), r, jnp.int32)])
            for k in range(DP // 16):
                rv[r, pl.ds(k * 16, 16)] = rv[r, pl.ds(k * 16, 16)] * a
        pltpu.async_copy(rv, acc_sh.at[dstp_v.at[j]], sems[slot], add=True)

    z = jnp.zeros((16,), jnp.float32)

    @plsc.parallel_loop(0, C, unroll=4)
    def _(r):
        for k in range(DP // 16):
            rows0[r, pl.ds(k * 16, 16)] = z
    base = sid * RPT
    nfull = RPT // C
    for t in range(nfull):
        pltpu.sync_copy(rows0, acc_sh.at[pl.ds(base + t * C, C)])
    rem = RPT - nfull * C
    if rem:
        pltpu.sync_copy(rows0.at[pl.ds(0, rem)],
                        acc_sh.at[pl.ds(base + nfull * C, rem)])
    plsc.subcore_barrier()

    for p in range(NPASS):
        phase = (p * CPP) % NBUF
        if p > 0:
            prev_phase = ((p - 1) * CPP) % NBUF
            wait_scatter((CPP - 1 + prev_phase) % NBUF)
        pltpu.sync_copy(src_hbm.at[wid, pl.ds(p * CPP, CPP)], srcp_v)
        pltpu.sync_copy(dst_hbm.at[wid, pl.ds(p * CPP, CPP)], dstp_v)
        issue_gathers(phase % NBUF, 0)
        issue_gathers((phase + 1) % NBUF, 1)

        def chunk(i, _):
            for m in range(NBUF):
                slot = (m + phase) % NBUF

                @pl.when(lax.rem(i, NBUF) == m)
                def _():
                    nxt = (slot + 2) % NBUF

                    @pl.when(i >= 1)
                    def _():
                        wait_scatter(nxt)

                    @pl.when(i + 2 < CPP)
                    def _():
                        issue_gathers(nxt, i + 2)

                    compute_and_scatter(slot, i)
            return 0

        lax.fori_loop(0, CPP, chunk, 0)

    wait_scatter((CPP - 1 + (NPASS - 1) * CPP) % NBUF)
    plsc.subcore_barrier()

    pltpu.sync_copy(acc_sh.at[pl.ds(base, RPT)],
                    out_hbm.at[cid, pl.ds(base, RPT)])


_sc_gat = pl.kernel(
    _sc_gat_body,
    out_type=jax.ShapeDtypeStruct((2, N, DP), jnp.float32),
    mesh=plsc.VectorSubcoreMesh(core_axis_name="c", subcore_axis_name="s"),
    compiler_params=pltpu.CompilerParams(use_tc_tiling_on_sc=False,
                                         needs_layout_passes=False),
    scratch_types=(
        [pltpu.VMEM((CPP, C), jnp.int32)] * 2 +
        [pltpu.VMEM((C, DP), jnp.float32)] * 3 +
        [pltpu.VMEM((C,), jnp.float32)] * 3 +
        [pltpu.VMEM((C,), jnp.float32)] +
        [pltpu.VMEM_SHARED((N, DP), jnp.float32)] +
        [pltpu.SemaphoreType.DMA] * 9
    ),
)


def kernel(x, edge_index, W1, al1, ar1, b1, W2, al2, ar2, b2):
    src = edge_index[0].astype(jnp.int32).reshape(NW, NCH, C)
    dst = edge_index[1].astype(jnp.int32).reshape(NW, NCH, C)

    featp1, er1 = _tc_feat(x, W1, al1.reshape(D, 1), ar1.reshape(D, 1))
    acc1 = _sc_gat(featp1, er1.reshape(N), src, dst)
    featp2, er2 = _tc_mid(acc1, b1.reshape(1, D), W2,
                          al2.reshape(D, 1), ar2.reshape(D, 1))
    acc2 = _sc_gat(featp2, er2.reshape(N), src, dst)
    return _tc_out(acc2, b2.reshape(1, D))


# speedup vs baseline: 1.3202x; 1.3202x over previous
import functools

import jax
import jax.numpy as jnp
from jax import lax
from jax.experimental import pallas as pl
from jax.experimental.pallas import tpu as pltpu
from jax.experimental.pallas import tpu_sc as plsc

N = 10000
E = 320000
D = 128
NW = 32
EPW = E // NW
C = 80
NCH = EPW // C
RPT = N // 16
DPT = 624
NPASS = 5
CPP = NCH // NPASS
NBUF = 3


def _proj(f, al, ar):
    el = jnp.sum(f * al, axis=1)
    er = jnp.sum(f * ar, axis=1)
    return el, er


def _tc_feat_body(x_ref, w_ref, al_ref, ar_ref, f_ref, el_ref, er_ref):
    f = jnp.dot(x_ref[...], w_ref[...], preferred_element_type=jnp.float32,
                precision=lax.Precision.HIGHEST)
    f_ref[...] = f
    el_ref[...], er_ref[...] = _proj(f, al_ref[...], ar_ref[...])


_tc_feat = pl.pallas_call(
    _tc_feat_body,
    out_shape=[
        jax.ShapeDtypeStruct((N, D), jnp.float32),
        jax.ShapeDtypeStruct((N,), jnp.float32),
        jax.ShapeDtypeStruct((N,), jnp.float32),
    ],
)


def _combine(accs_ref, dens_ref):
    a = accs_ref[0] + accs_ref[1]
    den = dens_ref[0] + dens_ref[1]
    den = jnp.where(den == 0.0, 1.0, den)
    return a * (1.0 / den).reshape(N, 1)


def _tc_mid_body(accs_ref, dens_ref, b_ref, w_ref, al_ref, ar_ref,
                 f_ref, el_ref, er_ref):
    h = jnp.maximum(_combine(accs_ref, dens_ref) + b_ref[...], 0.0)
    f = jnp.dot(h, w_ref[...], preferred_element_type=jnp.float32,
                precision=lax.Precision.HIGHEST)
    f_ref[...] = f
    el_ref[...], er_ref[...] = _proj(f, al_ref[...], ar_ref[...])


_tc_mid = pl.pallas_call(
    _tc_mid_body,
    out_shape=[
        jax.ShapeDtypeStruct((N, D), jnp.float32),
        jax.ShapeDtypeStruct((N,), jnp.float32),
        jax.ShapeDtypeStruct((N,), jnp.float32),
    ],
)


def _tc_out_body(accs_ref, dens_ref, b_ref, out_ref):
    out_ref[...] = _combine(accs_ref, dens_ref) + b_ref[...]


_tc_out = pl.pallas_call(
    _tc_out_body,
    out_shape=jax.ShapeDtypeStruct((N, D), jnp.float32),
)


def _sc_gat_body(feat_hbm, el_hbm, er_hbm, src_hbm, dst_hbm,
                 accs_hbm, dens_hbm,
                 srcp_v, dstp_v, rows0, rows1, rows2, els0, els1, els2,
                 ers0, ers1, ers2, ex0, ex1, ex2, zden_v,
                 acc_sh, den_sh,
                 semr0, semr1, semr2, sere0, sere1, sere2,
                 sems0, sems1, sems2):
    cid = lax.axis_index("c")
    sid = lax.axis_index("s")
    wid = cid * 16 + sid

    rows = [rows0, rows1, rows2]
    els = [els0, els1, els2]
    ers = [ers0, ers1, ers2]
    exs = [ex0, ex1, ex2]
    semr = [semr0, semr1, semr2]
    sere = [sere0, sere1, sere2]
    sems = [sems0, sems1, sems2]

    def issue_gathers(slot, j):
        pltpu.async_copy(el_hbm.at[srcp_v.at[j]], els[slot], sere[slot])
        pltpu.async_copy(er_hbm.at[dstp_v.at[j]], ers[slot], sere[slot])
        pltpu.async_copy(feat_hbm.at[srcp_v.at[j]], rows[slot], semr[slot])

    def wait_rows(slot):
        pltpu.make_async_copy(feat_hbm.at[pl.ds(0, C)], rows[slot],
                              semr[slot]).wait()

    def wait_logits(slot):
        pltpu.make_async_copy(el_hbm.at[pl.ds(0, C)], els[slot],
                              sere[slot]).wait()
        pltpu.make_async_copy(er_hbm.at[pl.ds(0, C)], ers[slot],
                              sere[slot]).wait()

    def wait_scatter(slot):
        pltpu.make_async_copy(feat_hbm.at[pl.ds(0, C)], rows[slot],
                              sems[slot]).wait()
        pltpu.make_async_copy(el_hbm.at[pl.ds(0, C)], exs[slot],
                              sems[slot]).wait()

    def compute_and_scatter(slot, j):
        wait_logits(slot)
        ex_v = exs[slot]
        for g in range(C // 16):
            e = els[slot][pl.ds(g * 16, 16)] + ers[slot][pl.ds(g * 16, 16)]
            e = jnp.maximum(e, 0.2 * e)
            ex_v[pl.ds(g * 16, 16)] = jnp.exp(e)
        wait_rows(slot)
        rv = rows[slot]

        @plsc.parallel_loop(0, C, unroll=2)
        def _(r):
            a = plsc.load_gather(ex_v, [jnp.full((16,), r, jnp.int32)])
            for k in range(D // 16):
                rv[r, pl.ds(k * 16, 16)] = rv[r, pl.ds(k * 16, 16)] * a

        pltpu.async_copy(rv, acc_sh.at[dstp_v.at[j]], sems[slot], add=True)
        pltpu.async_copy(ex_v, den_sh.at[dstp_v.at[j]], sems[slot], add=True)

    z = jnp.zeros((16,), jnp.float32)

    @plsc.parallel_loop(0, C, unroll=4)
    def _(r):
        for k in range(D // 16):
            rows0[r, pl.ds(k * 16, 16)] = z

    @plsc.parallel_loop(0, 640 // 16, unroll=4)
    def _(r):
        zden_v[pl.ds(r * 16, 16)] = z

    base = sid * RPT
    nfull = RPT // C
    for t in range(nfull):
        pltpu.sync_copy(rows0, acc_sh.at[pl.ds(base + t * C, C)])
    rem = RPT - nfull * C
    if rem:
        pltpu.sync_copy(rows0.at[pl.ds(0, rem)],
                        acc_sh.at[pl.ds(base + nfull * C, rem)])
    dbase = sid * DPT
    pltpu.sync_copy(zden_v.at[pl.ds(0, DPT)], den_sh.at[pl.ds(dbase, DPT)])

    @pl.when(sid == 15)
    def _():
        pltpu.sync_copy(zden_v.at[pl.ds(0, 16)],
                        den_sh.at[pl.ds(16 * DPT, 16)])

    plsc.subcore_barrier()

    for p in range(NPASS):
        phase = (p * CPP) % NBUF
        if p > 0:
            prev_phase = ((p - 1) * CPP) % NBUF
            wait_scatter((CPP - 1 + prev_phase) % NBUF)
        pltpu.sync_copy(src_hbm.at[wid, pl.ds(p * CPP, CPP)], srcp_v)
        pltpu.sync_copy(dst_hbm.at[wid, pl.ds(p * CPP, CPP)], dstp_v)
        issue_gathers(phase % NBUF, 0)
        issue_gathers((phase + 1) % NBUF, 1)

        def chunk(i, _):
            for m in range(NBUF):
                slot = (m + phase) % NBUF

                @pl.when(lax.rem(i, NBUF) == m)
                def _():
                    nxt = (slot + 2) % NBUF

                    @pl.when(i >= 1)
                    def _():
                        wait_scatter(nxt)

                    @pl.when(i + 2 < CPP)
                    def _():
                        issue_gathers(nxt, i + 2)

                    compute_and_scatter(slot, i)
            return 0

        lax.fori_loop(0, CPP, chunk, 0)

    wait_scatter((CPP - 1 + (NPASS - 1) * CPP) % NBUF)
    plsc.subcore_barrier()

    pltpu.sync_copy(acc_sh.at[pl.ds(base, RPT)],
                    accs_hbm.at[cid, pl.ds(base, RPT)])
    pltpu.sync_copy(den_sh.at[pl.ds(dbase, DPT)],
                    dens_hbm.at[cid, pl.ds(dbase, DPT)])

    @pl.when(sid == 15)
    def _():
        pltpu.sync_copy(den_sh.at[pl.ds(16 * DPT, 16)],
                        dens_hbm.at[cid, pl.ds(16 * DPT, 16)])


_sc_gat = pl.kernel(
    _sc_gat_body,
    out_type=[
        jax.ShapeDtypeStruct((2, N, D), jnp.float32),
        jax.ShapeDtypeStruct((2, N), jnp.float32),
    ],
    mesh=plsc.VectorSubcoreMesh(core_axis_name="c", subcore_axis_name="s"),
    compiler_params=pltpu.CompilerParams(use_tc_tiling_on_sc=False,
                                         needs_layout_passes=False),
    scratch_types=(
        [pltpu.VMEM((CPP, C), jnp.int32)] * 2 +
        [pltpu.VMEM((C, D), jnp.float32)] * 3 +
        [pltpu.VMEM((C,), jnp.float32)] * 3 +
        [pltpu.VMEM((C,), jnp.float32)] * 3 +
        [pltpu.VMEM((C,), jnp.float32)] * 3 +
        [pltpu.VMEM((640,), jnp.float32)] +
        [pltpu.VMEM_SHARED((N, D), jnp.float32)] +
        [pltpu.VMEM_SHARED((N,), jnp.float32)] +
        [pltpu.SemaphoreType.DMA] * 9
    ),
)


def kernel(x, edge_index, W1, al1, ar1, b1, W2, al2, ar2, b2):
    src = edge_index[0].astype(jnp.int32).reshape(NW, NCH, C)
    dst = edge_index[1].astype(jnp.int32).reshape(NW, NCH, C)

    f1, el1, er1 = _tc_feat(x, W1, al1.reshape(1, D), ar1.reshape(1, D))
    accs1, dens1 = _sc_gat(f1, el1, er1, src, dst)
    f2, el2, er2 = _tc_mid(accs1, dens1, b1.reshape(1, D), W2,
                           al2.reshape(1, D), ar2.reshape(1, D))
    accs2, dens2 = _sc_gat(f2, el2, er2, src, dst)
    return _tc_out(accs2, dens2, b2.reshape(1, D))
